# baseline (speedup 1.0000x reference)
import functools

import jax
import jax.numpy as jnp
from jax import lax
from jax.experimental import pallas as pl
from jax.experimental.pallas import tpu as pltpu
from jax.experimental.pallas import tpu_sc as plsc

NC = 2
NS = 16
NW = NC * NS

IDX_W = 128
CHUNK = 256


def _mlp_table_body(emb_ref, wf_ref, bf_ref, wo_ref, ls_ref, lb_ref, out_ref):
    h = jnp.dot(emb_ref[...], wf_ref[...], preferred_element_type=jnp.float32)
    h = h + bf_ref[...]
    h = jax.nn.gelu(h, approximate=True)
    h = jnp.dot(h, wo_ref[...], preferred_element_type=jnp.float32)
    mu = jnp.mean(h, axis=-1, keepdims=True)
    d = h - mu
    var = jnp.mean(d * d, axis=-1, keepdims=True)
    h = d * lax.rsqrt(var + 1e-6)
    out_ref[...] = h * ls_ref[...] + lb_ref[...]


def _compute_table(embedding, W_fuse, b_fuse, W_out, ln_scale, ln_bias):
    vocab = embedding.shape[0]
    hidden = W_out.shape[1]
    return pl.pallas_call(
        _mlp_table_body,
        out_shape=jax.ShapeDtypeStruct((vocab, hidden), jnp.float32),
    )(embedding, W_fuse, b_fuse.reshape(1, hidden), W_out,
      ln_scale.reshape(1, hidden), ln_bias.reshape(1, hidden))


def _sc_gather(table, idx2d, batch, n, hidden):
    hs = n // 2
    vocab = table.shape[0]
    slabs = batch * n
    slabs_w = slabs // NW
    mesh = plsc.VectorSubcoreMesh(core_axis_name="c", subcore_axis_name="s")

    @functools.partial(
        pl.kernel,
        mesh=mesh,
        out_type=jax.ShapeDtypeStruct((batch, n, n, hidden), jnp.float32),
        scratch_types=[
            pltpu.VMEM_SHARED((vocab, hidden), jnp.float32),
            pltpu.VMEM((slabs_w * n,), jnp.int32),
            pltpu.VMEM((n, hidden), jnp.float32),
            pltpu.VMEM((n, hidden), jnp.float32),
            pltpu.SemaphoreType.DMA,
            pltpu.SemaphoreType.DMA,
            pltpu.SemaphoreType.DMA,
            pltpu.SemaphoreType.DMA,
        ],
    )
    def k(table_hbm, idx_hbm, out_hbm, table_s, idx_v, rows0, rows1,
          gsem0, gsem1, ssem0, ssem1):
        sid = lax.axis_index("s")
        wid = sid * NC + lax.axis_index("c")
        @pl.when(sid == 0)
        def _():
            pltpu.sync_copy(table_hbm, table_s)
        plsc.subcore_barrier()

        bq = wid // (n // slabs_w)
        i0 = (wid % (n // slabs_w)) * slabs_w
        pltpu.sync_copy(idx_hbm.at[pl.ds(wid * slabs_w * n, slabs_w * n)],
                        idx_v)

        rows = (rows0, rows1)
        gsem = (gsem0, gsem1)
        ssem = (ssem0, ssem1)

        def gather_slab(t, b):
            cps = [
                pltpu.async_copy(
                    table_s.at[idx_v.at[pl.ds(t * n + j * hs, hs)]],
                    rows[b].at[pl.ds(j * hs, hs)],
                    gsem[b],
                )
                for j in range(2)
            ]
            for cp in cps:
                cp.wait()

        def start_store(t, b):
            pltpu.async_copy(rows[b], out_hbm.at[bq, i0 + t], ssem[b])

        def wait_store(b):
            pltpu.make_async_copy(rows[b], out_hbm.at[bq, i0], ssem[b]).wait()

        for b in range(2):
            gather_slab(b, b)
            start_store(b, b)

        @pl.loop(2, slabs_w, step=2)
        def _(t):
            for b in range(2):
                wait_store(b)
                gather_slab(t + b, b)
                start_store(t + b, b)

        for b in range(2):
            wait_store(b)

    return k(table, idx2d)


def kernel(edge_types, embedding, W_fuse, b_fuse, W_out, ln_scale, ln_bias):
    batch, n, _ = edge_types.shape
    hidden = W_out.shape[1]
    total = batch * n * n
    table = _compute_table(embedding, W_fuse, b_fuse, W_out, ln_scale, ln_bias)
    idx_flat = edge_types.reshape(total)
    return _sc_gather(table, idx_flat, batch, n, hidden)


# speedup vs baseline: 1.2734x; 1.2734x over previous
import functools

import jax
import jax.numpy as jnp
from jax import lax
from jax.experimental import pallas as pl
from jax.experimental.pallas import tpu as pltpu
from jax.experimental.pallas import tpu_sc as plsc

NC = 2
NS = 16
NW = NC * NS

IDX_W = 128
CHUNK = 256


def _mlp_table_body(emb_ref, wf_ref, bf_ref, wo_ref, ls_ref, lb_ref, out_ref):
    h = jnp.dot(emb_ref[...], wf_ref[...], preferred_element_type=jnp.float32)
    h = h + bf_ref[...]
    h = jax.nn.gelu(h, approximate=True)
    h = jnp.dot(h, wo_ref[...], preferred_element_type=jnp.float32)
    mu = jnp.mean(h, axis=-1, keepdims=True)
    d = h - mu
    var = jnp.mean(d * d, axis=-1, keepdims=True)
    h = d * lax.rsqrt(var + 1e-6)
    out_ref[...] = h * ls_ref[...] + lb_ref[...]


def _compute_table(embedding, W_fuse, b_fuse, W_out, ln_scale, ln_bias):
    vocab = embedding.shape[0]
    hidden = W_out.shape[1]
    return pl.pallas_call(
        _mlp_table_body,
        out_shape=jax.ShapeDtypeStruct((vocab, hidden), jnp.float32),
    )(embedding, W_fuse, b_fuse.reshape(1, hidden), W_out,
      ln_scale.reshape(1, hidden), ln_bias.reshape(1, hidden))


def _sc_gather(table, idx_flat, total, hidden):
    vocab = table.shape[0]
    per_w = total // NW
    nchunk = per_w // CHUNK
    gpc = CHUNK // IDX_W
    mesh = plsc.VectorSubcoreMesh(core_axis_name="c", subcore_axis_name="s")

    @functools.partial(
        pl.kernel,
        mesh=mesh,
        out_type=jax.ShapeDtypeStruct((total, hidden), jnp.float32),
        scratch_types=[
            pltpu.VMEM_SHARED((vocab, hidden), jnp.float32),
            pltpu.VMEM((per_w,), jnp.int32),
            pltpu.VMEM((CHUNK, hidden), jnp.float32),
            pltpu.VMEM((CHUNK, hidden), jnp.float32),
            pltpu.SemaphoreType.DMA,
            pltpu.SemaphoreType.DMA,
            pltpu.SemaphoreType.DMA,
            pltpu.SemaphoreType.DMA,
        ],
    )
    def k(table_hbm, idx_hbm, out_hbm, table_s, idx_v, rows0, rows1,
          gsem0, gsem1, ssem0, ssem1):
        sid = lax.axis_index("s")
        wid = sid * NC + lax.axis_index("c")
        @pl.when(sid == 0)
        def _():
            pltpu.sync_copy(table_hbm, table_s)
        plsc.subcore_barrier()

        base = wid * per_w
        pltpu.sync_copy(idx_hbm.at[pl.ds(base, per_w)], idx_v)

        rows = (rows0, rows1)
        gsem = (gsem0, gsem1)
        ssem = (ssem0, ssem1)

        def gather_chunk(c, b):
            cps = [
                pltpu.async_copy(
                    table_s.at[idx_v.at[pl.ds(c * CHUNK + j * IDX_W, IDX_W)]],
                    rows[b].at[pl.ds(j * IDX_W, IDX_W)],
                    gsem[b],
                )
                for j in range(gpc)
            ]
            for cp in cps:
                cp.wait()

        def start_store(c, b):
            pltpu.async_copy(
                rows[b], out_hbm.at[pl.ds(base + c * CHUNK, CHUNK)], ssem[b])

        def wait_store(b):
            pltpu.make_async_copy(
                rows[b], out_hbm.at[pl.ds(base, CHUNK)], ssem[b]).wait()

        for b in range(2):
            gather_chunk(b, b)
            start_store(b, b)

        @pl.loop(2, nchunk, step=2)
        def _(c):
            for b in range(2):
                wait_store(b)
                gather_chunk(c + b, b)
                start_store(c + b, b)

        for b in range(2):
            wait_store(b)

    return k(table, idx_flat)


def kernel(edge_types, embedding, W_fuse, b_fuse, W_out, ln_scale, ln_bias):
    batch, n, _ = edge_types.shape
    hidden = W_out.shape[1]
    total = batch * n * n
    table = _compute_table(embedding, W_fuse, b_fuse, W_out, ln_scale, ln_bias)
    idx_flat = edge_types.reshape(total)
    out = _sc_gather(table, idx_flat, total, hidden)
    return out.reshape(batch, n, n, hidden)
